# baseline (speedup 1.0000x reference)
import functools

import jax
import jax.numpy as jnp
from jax import lax
from jax.experimental import pallas as pl
from jax.experimental.pallas import tpu as pltpu
from jax.experimental.pallas import tpu_sc as plsc

N = 10000
E = 320000
HID = 32
NC, NS = 2, 16
NW = NC * NS
CHUNK = 128
BLK = 4
NBUF = 2
MCW = -(-E // (NW * CHUNK * BLK))
CPW = MCW * BLK
EPW = CPW * CHUNK
E_PAD = NW * EPW
NPAD = 10240
RPT = NPAD // NS
SINK = N

_mesh = plsc.VectorSubcoreMesh(core_axis_name="c", subcore_axis_name="s")
_sc_params = pltpu.CompilerParams(use_tc_tiling_on_sc=False)


@functools.partial(
    pl.kernel,
    out_type=jax.ShapeDtypeStruct((NC, NPAD, 16), jnp.float32),
    mesh=_mesh,
    scratch_types=[
        pltpu.VMEM((MCW, BLK * CHUNK), jnp.int32),
        pltpu.VMEM((BLK * CHUNK, 16), jnp.float32),
        pltpu.VMEM_SHARED((NPAD, 16), jnp.float32),
    ],
    compiler_params=_sc_params,
)
def _deg_kernel(dst_hbm, out_hbm, dstbuf, buf, deg_sh):
    c = lax.axis_index("c")
    s = lax.axis_index("s")
    wid = s * NC + c
    pltpu.sync_copy(dst_hbm.at[wid], dstbuf)

    def _zfill(i, _):
        buf[i, :] = jnp.zeros((16,), jnp.float32)
        return 0
    lax.fori_loop(0, BLK * CHUNK, _zfill, 0)
    for k in range(RPT // (BLK * CHUNK)):
        pltpu.sync_copy(buf, deg_sh.at[pl.ds(s * RPT + k * BLK * CHUNK,
                                             BLK * CHUNK)])

    def _ofill(i, _):
        buf[i, :] = jnp.ones((16,), jnp.float32)
        return 0
    lax.fori_loop(0, BLK * CHUNK, _ofill, 0)
    plsc.subcore_barrier()

    def _scat(j, _):
        pltpu.sync_copy(buf, deg_sh.at[dstbuf.at[j]], add=True)
        return 0
    lax.fori_loop(0, MCW, _scat, 0)
    plsc.subcore_barrier()
    pltpu.sync_copy(deg_sh.at[pl.ds(s * RPT, RPT)],
                    out_hbm.at[c, pl.ds(s * RPT, RPT)])


@functools.partial(
    pl.kernel,
    out_type=jax.ShapeDtypeStruct((NC, NPAD, HID), jnp.float32),
    mesh=_mesh,
    scratch_types=[
        pltpu.VMEM((MCW, BLK * CHUNK), jnp.int32),
        pltpu.VMEM((MCW, BLK * CHUNK), jnp.int32),
        pltpu.VMEM((NBUF, BLK * CHUNK, HID), jnp.float32),
        pltpu.VMEM((CHUNK, HID), jnp.float32),
        pltpu.VMEM_SHARED((NPAD, HID), jnp.float32),
    ] + [pltpu.SemaphoreType.DMA] * NBUF,
    compiler_params=_sc_params,
)
def _agg_kernel(y_hbm, src_hbm, dst_hbm, out_hbm,
                srcbuf, dstbuf, rows, zbuf, agg_sh, *sems):
    gsem = sems
    c = lax.axis_index("c")
    s = lax.axis_index("s")
    wid = s * NC + c
    pltpu.sync_copy(src_hbm.at[wid], srcbuf)
    pltpu.sync_copy(dst_hbm.at[wid], dstbuf)

    def _zfill(i, _):
        zbuf[i, pl.ds(0, 16)] = jnp.zeros((16,), jnp.float32)
        zbuf[i, pl.ds(16, 16)] = jnp.zeros((16,), jnp.float32)
        return 0
    lax.fori_loop(0, CHUNK, _zfill, 0)
    for k in range(RPT // CHUNK):
        pltpu.sync_copy(zbuf, agg_sh.at[pl.ds(s * RPT + k * CHUNK, CHUNK)])
    plsc.subcore_barrier()

    T = MCW // NBUF

    def _g(j, b):
        pltpu.async_copy(y_hbm.at[srcbuf.at[j]], rows.at[b], gsem[b])

    def _wait_g(j, b):
        pltpu.make_async_copy(y_hbm.at[srcbuf.at[j]], rows.at[b],
                              gsem[b]).wait()

    _g(0, 0)

    def _step(t, _):
        j0 = NBUF * t
        _g(j0 + 1, 1)
        _wait_g(j0, 0)
        pltpu.sync_copy(rows.at[0], agg_sh.at[dstbuf.at[j0]], add=True)
        j1 = j0 + 1

        @pl.when(t < T - 1)
        def _():
            _g(j1 + 1, 0)
        _wait_g(j1, 1)
        pltpu.sync_copy(rows.at[1], agg_sh.at[dstbuf.at[j1]], add=True)
        return 0
    lax.fori_loop(0, T, _step, 0)
    plsc.subcore_barrier()
    pltpu.sync_copy(agg_sh.at[pl.ds(s * RPT, RPT)],
                    out_hbm.at[c, pl.ds(s * RPT, RPT)])


def _tc_a_body(x_ref, w1_ref, d0_ref, d1_ref, y1_ref, dinv_ref):
    deg = d0_ref[...] + d1_ref[...] + 1.0
    dinv = lax.rsqrt(deg)
    xw = jnp.dot(x_ref[...], w1_ref[...], preferred_element_type=jnp.float32)
    y1_ref[...] = xw * dinv
    dinv_ref[...] = dinv


_tc_a = pl.pallas_call(
    _tc_a_body,
    out_shape=[jax.ShapeDtypeStruct((N, HID), jnp.float32),
               jax.ShapeDtypeStruct((N, 1), jnp.float32)],
)


def _tc_b_body(p0_ref, p1_ref, y1_ref, dinv_ref, b1_ref, w2_ref, y2_ref):
    dinv = dinv_ref[...]
    pre = dinv * (p0_ref[...] + p1_ref[...] + y1_ref[...]) + b1_ref[...]
    h1 = jnp.maximum(pre, 0.0)
    xw2 = jnp.dot(h1, w2_ref[...], preferred_element_type=jnp.float32)
    y2_ref[...] = xw2 * dinv


_tc_b = pl.pallas_call(
    _tc_b_body,
    out_shape=jax.ShapeDtypeStruct((N, HID), jnp.float32),
)


def _tc_c_body(p0_ref, p1_ref, y2_ref, dinv_ref, b2_ref, wout_ref, bout_ref,
               out_ref):
    dinv = dinv_ref[...]
    pre = dinv * (p0_ref[...] + p1_ref[...] + y2_ref[...]) + b2_ref[...]
    h2 = jnp.maximum(pre, 0.0)
    z = jnp.dot(h2, wout_ref[...], preferred_element_type=jnp.float32)
    out_ref[...] = jax.nn.sigmoid(z + bout_ref[...])


_tc_c = pl.pallas_call(
    _tc_c_body,
    out_shape=jax.ShapeDtypeStruct((N, 1), jnp.float32),
)


def kernel(x, edge_index, W1, b1, W2, b2, Wout, bout):
    ei = edge_index.astype(jnp.int32)
    src = jnp.concatenate([ei[0], jnp.zeros((E_PAD - E,), jnp.int32)])
    dst = jnp.concatenate([ei[1], jnp.full((E_PAD - E,), SINK, jnp.int32)])
    src3 = src.reshape(NW, MCW, BLK * CHUNK)
    dst3 = dst.reshape(NW, MCW, BLK * CHUNK)

    degp = _deg_kernel(dst3)
    d0 = degp[0, :N, 0:1]
    d1 = degp[1, :N, 0:1]
    y1, dinv = _tc_a(x, W1, d0, d1)

    p1 = _agg_kernel(y1, src3, dst3)
    y2 = _tc_b(p1[0, :N], p1[1, :N], y1, dinv, b1.reshape(1, HID), W2)

    p2 = _agg_kernel(y2, src3, dst3)
    out = _tc_c(p2[0, :N], p2[1, :N], y2, dinv, b2.reshape(1, HID),
                Wout, bout.reshape(1, 1))
    return out


# speedup vs baseline: 1.5966x; 1.5966x over previous
import functools

import jax
import jax.numpy as jnp
from jax import lax
from jax.experimental import pallas as pl
from jax.experimental.pallas import tpu as pltpu
from jax.experimental.pallas import tpu_sc as plsc

N = 10000
E = 320000
HID = 32
NC, NS = 2, 16
NW = NC * NS
CHUNK = 128
CPW = -(-E // (NW * CHUNK))
EPW = CPW * CHUNK
E_PAD = NW * EPW
NPAD = 10240
RPT = NPAD // NS
SINK = N

_mesh = plsc.VectorSubcoreMesh(core_axis_name="c", subcore_axis_name="s")
_sc_params = pltpu.CompilerParams(use_tc_tiling_on_sc=False)


@functools.partial(
    pl.kernel,
    out_type=jax.ShapeDtypeStruct((NC, NPAD, 16), jnp.float32),
    mesh=_mesh,
    scratch_types=[
        pltpu.VMEM((CPW, CHUNK), jnp.int32),
        pltpu.VMEM((CHUNK, 16), jnp.float32),
        pltpu.VMEM_SHARED((NPAD, 16), jnp.float32),
    ],
    compiler_params=_sc_params,
)
def _deg_kernel(dst_hbm, out_hbm, dstbuf, buf, deg_sh):
    c = lax.axis_index("c")
    s = lax.axis_index("s")
    wid = s * NC + c
    pltpu.sync_copy(dst_hbm.at[wid], dstbuf)

    def _zfill(i, _):
        buf[i, :] = jnp.zeros((16,), jnp.float32)
        return 0
    lax.fori_loop(0, CHUNK, _zfill, 0)
    for k in range(RPT // CHUNK):
        pltpu.sync_copy(buf, deg_sh.at[pl.ds(s * RPT + k * CHUNK, CHUNK)])

    def _ofill(i, _):
        buf[i, :] = jnp.ones((16,), jnp.float32)
        return 0
    lax.fori_loop(0, CHUNK, _ofill, 0)
    plsc.subcore_barrier()

    def _scat(j, _):
        pltpu.sync_copy(buf, deg_sh.at[dstbuf.at[j]], add=True)
        return 0
    lax.fori_loop(0, CPW, _scat, 0)
    plsc.subcore_barrier()
    pltpu.sync_copy(deg_sh.at[pl.ds(s * RPT, RPT)],
                    out_hbm.at[c, pl.ds(s * RPT, RPT)])


@functools.partial(
    pl.kernel,
    out_type=jax.ShapeDtypeStruct((NC, NPAD, HID), jnp.float32),
    mesh=_mesh,
    scratch_types=[
        pltpu.VMEM((CPW, CHUNK), jnp.int32),
        pltpu.VMEM((CPW, CHUNK), jnp.int32),
        pltpu.VMEM((CHUNK, HID), jnp.float32),
        pltpu.VMEM((CHUNK, HID), jnp.float32),
        pltpu.VMEM((RPT, HID), jnp.float32),
        pltpu.VMEM_SHARED((NPAD, HID), jnp.float32),
        pltpu.VMEM_SHARED((NPAD, HID), jnp.float32),
        pltpu.SemaphoreType.DMA,
    ],
    compiler_params=_sc_params,
)
def _agg_kernel(y_hbm, src_hbm, dst_hbm, out_hbm,
                srcbuf, dstbuf, rows, zbuf, ybuf, y_sh, agg_sh, sem):
    c = lax.axis_index("c")
    s = lax.axis_index("s")
    wid = s * NC + c
    pltpu.sync_copy(src_hbm.at[wid], srcbuf)
    pltpu.sync_copy(dst_hbm.at[wid], dstbuf)

    def _zfill(i, _):
        zbuf[i, pl.ds(0, 16)] = jnp.zeros((16,), jnp.float32)
        zbuf[i, pl.ds(16, 16)] = jnp.zeros((16,), jnp.float32)
        return 0
    lax.fori_loop(0, CHUNK, _zfill, 0)
    for k in range(RPT // CHUNK):
        pltpu.sync_copy(zbuf, agg_sh.at[pl.ds(s * RPT + k * CHUNK, CHUNK)])
    pltpu.sync_copy(y_hbm.at[pl.ds(s * RPT, RPT)], ybuf)
    pltpu.sync_copy(ybuf, y_sh.at[pl.ds(s * RPT, RPT)])
    plsc.subcore_barrier()

    def _step(j, _):
        pltpu.async_copy(y_sh.at[srcbuf.at[j]], rows, sem).wait()
        pltpu.sync_copy(rows, agg_sh.at[dstbuf.at[j]], add=True)
        return 0
    lax.fori_loop(0, CPW, _step, 0)
    plsc.subcore_barrier()
    pltpu.sync_copy(agg_sh.at[pl.ds(s * RPT, RPT)],
                    out_hbm.at[c, pl.ds(s * RPT, RPT)])


def _tc_a_body(x_ref, w1_ref, d0_ref, d1_ref, y1_ref, dinv_ref):
    deg = d0_ref[...] + d1_ref[...] + 1.0
    dinv = lax.rsqrt(deg)
    xw = jnp.dot(x_ref[...], w1_ref[...], preferred_element_type=jnp.float32)
    y1_ref[...] = xw * dinv
    dinv_ref[...] = dinv


_tc_a = pl.pallas_call(
    _tc_a_body,
    out_shape=[jax.ShapeDtypeStruct((N, HID), jnp.float32),
               jax.ShapeDtypeStruct((N, 1), jnp.float32)],
)


def _tc_b_body(p0_ref, p1_ref, y1_ref, dinv_ref, b1_ref, w2_ref, y2_ref):
    dinv = dinv_ref[...]
    pre = dinv * (p0_ref[...] + p1_ref[...] + y1_ref[...]) + b1_ref[...]
    h1 = jnp.maximum(pre, 0.0)
    xw2 = jnp.dot(h1, w2_ref[...], preferred_element_type=jnp.float32)
    y2_ref[...] = xw2 * dinv


_tc_b = pl.pallas_call(
    _tc_b_body,
    out_shape=jax.ShapeDtypeStruct((N, HID), jnp.float32),
)


def _tc_c_body(p0_ref, p1_ref, y2_ref, dinv_ref, b2_ref, wout_ref, bout_ref,
               out_ref):
    dinv = dinv_ref[...]
    pre = dinv * (p0_ref[...] + p1_ref[...] + y2_ref[...]) + b2_ref[...]
    h2 = jnp.maximum(pre, 0.0)
    z = jnp.dot(h2, wout_ref[...], preferred_element_type=jnp.float32)
    out_ref[...] = jax.nn.sigmoid(z + bout_ref[...])


_tc_c = pl.pallas_call(
    _tc_c_body,
    out_shape=jax.ShapeDtypeStruct((N, 1), jnp.float32),
)


def kernel(x, edge_index, W1, b1, W2, b2, Wout, bout):
    ei = edge_index.astype(jnp.int32)
    src = jnp.concatenate([ei[0], jnp.zeros((E_PAD - E,), jnp.int32)])
    dst = jnp.concatenate([ei[1], jnp.full((E_PAD - E,), SINK, jnp.int32)])
    src3 = src.reshape(NW, CPW, CHUNK)
    dst3 = dst.reshape(NW, CPW, CHUNK)

    degp = _deg_kernel(dst3)
    d0 = degp[0, :N, 0:1]
    d1 = degp[1, :N, 0:1]
    y1, dinv = _tc_a(x, W1, d0, d1)

    y1p = jnp.pad(y1, ((0, NPAD - N), (0, 0)))
    p1 = _agg_kernel(y1p, src3, dst3)
    y2 = _tc_b(p1[0, :N], p1[1, :N], y1, dinv, b1.reshape(1, HID), W2)

    y2p = jnp.pad(y2, ((0, NPAD - N), (0, 0)))
    p2 = _agg_kernel(y2p, src3, dst3)
    out = _tc_c(p2[0, :N], p2[1, :N], y2, dinv, b2.reshape(1, HID),
                Wout, bout.reshape(1, 1))
    return out


# speedup vs baseline: 1.7407x; 1.0903x over previous
import functools

import jax
import jax.numpy as jnp
from jax import lax
from jax.experimental import pallas as pl
from jax.experimental.pallas import tpu as pltpu
from jax.experimental.pallas import tpu_sc as plsc

N = 10000
E = 320000
HID = 32
NC, NS = 2, 16
NW = NC * NS
CHUNK = 128
CPW = -(-E // (NW * CHUNK))
EPW = CPW * CHUNK
E_PAD = NW * EPW
NPAD = 10240
RPT = NPAD // NS
SINK = N

_mesh = plsc.VectorSubcoreMesh(core_axis_name="c", subcore_axis_name="s")
_sc_params = pltpu.CompilerParams(use_tc_tiling_on_sc=False)


@functools.partial(
    pl.kernel,
    out_type=jax.ShapeDtypeStruct((NC, NPAD, 16), jnp.float32),
    mesh=_mesh,
    scratch_types=[
        pltpu.VMEM((CPW, CHUNK), jnp.int32),
        pltpu.VMEM((CHUNK, 16), jnp.float32),
        pltpu.VMEM_SHARED((NPAD, 16), jnp.float32),
    ],
    compiler_params=_sc_params,
)
def _deg_kernel(dst_hbm, out_hbm, dstbuf, buf, deg_sh):
    c = lax.axis_index("c")
    s = lax.axis_index("s")
    wid = s * NC + c
    pltpu.sync_copy(dst_hbm.at[wid], dstbuf)

    def _zfill(i, _):
        buf[i, :] = jnp.zeros((16,), jnp.float32)
        return 0
    lax.fori_loop(0, CHUNK, _zfill, 0)
    for k in range(RPT // CHUNK):
        pltpu.sync_copy(buf, deg_sh.at[pl.ds(s * RPT + k * CHUNK, CHUNK)])

    def _ofill(i, _):
        buf[i, :] = jnp.ones((16,), jnp.float32)
        return 0
    lax.fori_loop(0, CHUNK, _ofill, 0)
    plsc.subcore_barrier()

    def _scat(j, _):
        pltpu.sync_copy(buf, deg_sh.at[dstbuf.at[j]], add=True)
        return 0
    lax.fori_loop(0, CPW, _scat, 0)
    plsc.subcore_barrier()
    pltpu.sync_copy(deg_sh.at[pl.ds(s * RPT, RPT)],
                    out_hbm.at[c, pl.ds(s * RPT, RPT)])


@functools.partial(
    pl.kernel,
    out_type=jax.ShapeDtypeStruct((NC, NPAD, HID), jnp.float32),
    mesh=_mesh,
    scratch_types=[
        pltpu.VMEM((CPW, CHUNK), jnp.int32),
        pltpu.VMEM((CPW, CHUNK), jnp.int32),
        pltpu.VMEM((CHUNK, HID), jnp.float32),
        pltpu.VMEM((CHUNK, HID), jnp.float32),
        pltpu.VMEM((RPT, HID), jnp.float32),
        pltpu.VMEM_SHARED((NPAD, HID), jnp.float32),
        pltpu.VMEM_SHARED((NPAD, HID), jnp.float32),
        pltpu.SemaphoreType.DMA,
    ],
    compiler_params=_sc_params,
)
def _agg_kernel(y_hbm, src_hbm, dst_hbm, out_hbm,
                srcbuf, dstbuf, rows, zbuf, ybuf, y_sh, agg_sh, sem):
    c = lax.axis_index("c")
    s = lax.axis_index("s")
    wid = s * NC + c
    pltpu.sync_copy(src_hbm.at[wid], srcbuf)
    pltpu.sync_copy(dst_hbm.at[wid], dstbuf)

    def _zfill(i, _):
        zbuf[i, pl.ds(0, 16)] = jnp.zeros((16,), jnp.float32)
        zbuf[i, pl.ds(16, 16)] = jnp.zeros((16,), jnp.float32)
        return 0
    lax.fori_loop(0, CHUNK, _zfill, 0)
    for k in range(RPT // CHUNK):
        pltpu.sync_copy(zbuf, agg_sh.at[pl.ds(s * RPT + k * CHUNK, CHUNK)])
    pltpu.sync_copy(y_hbm.at[pl.ds(s * RPT, RPT)], ybuf)
    pltpu.sync_copy(ybuf, y_sh.at[pl.ds(s * RPT, RPT)])
    plsc.subcore_barrier()

    def _step(j, _):
        pltpu.async_copy(y_sh.at[srcbuf.at[j]], rows, sem).wait()
        pltpu.sync_copy(rows, agg_sh.at[dstbuf.at[j]], add=True)
        return 0
    lax.fori_loop(0, CPW, _step, 0)
    plsc.subcore_barrier()
    pltpu.sync_copy(agg_sh.at[pl.ds(s * RPT, RPT)],
                    out_hbm.at[c, pl.ds(s * RPT, RPT)])


def _tc_mm_body(x_ref, w1_ref, xw_ref):
    xw_ref[...] = jnp.dot(x_ref[...], w1_ref[...],
                          preferred_element_type=jnp.float32)


_tc_mm = pl.pallas_call(
    _tc_mm_body,
    out_shape=jax.ShapeDtypeStruct((N, HID), jnp.float32),
)


def _tc_a_body(xw_ref, degp_ref, y1_ref, dinv_ref):
    deg = degp_ref[0, :N, 0:1] + degp_ref[1, :N, 0:1] + 1.0
    dinv = lax.rsqrt(deg)
    y1_ref[pl.ds(0, N)] = xw_ref[...] * dinv
    y1_ref[pl.ds(N, NPAD - N)] = jnp.zeros((NPAD - N, HID), jnp.float32)
    dinv_ref[...] = dinv


_tc_a = pl.pallas_call(
    _tc_a_body,
    out_shape=[jax.ShapeDtypeStruct((NPAD, HID), jnp.float32),
               jax.ShapeDtypeStruct((N, 1), jnp.float32)],
)


def _tc_b_body(p_ref, y1_ref, dinv_ref, b1_ref, w2_ref, y2_ref):
    dinv = dinv_ref[...]
    agg = p_ref[0, :N, :] + p_ref[1, :N, :] + y1_ref[pl.ds(0, N)]
    h1 = jnp.maximum(dinv * agg + b1_ref[...], 0.0)
    xw2 = jnp.dot(h1, w2_ref[...], preferred_element_type=jnp.float32)
    y2_ref[pl.ds(0, N)] = xw2 * dinv
    y2_ref[pl.ds(N, NPAD - N)] = jnp.zeros((NPAD - N, HID), jnp.float32)


_tc_b = pl.pallas_call(
    _tc_b_body,
    out_shape=jax.ShapeDtypeStruct((NPAD, HID), jnp.float32),
)


def _tc_c_body(p_ref, y2_ref, dinv_ref, b2_ref, wout_ref, bout_ref,
               out_ref):
    dinv = dinv_ref[...]
    agg = p_ref[0, :N, :] + p_ref[1, :N, :] + y2_ref[pl.ds(0, N)]
    h2 = jnp.maximum(dinv * agg + b2_ref[...], 0.0)
    z = jnp.dot(h2, wout_ref[...], preferred_element_type=jnp.float32)
    out_ref[...] = jax.nn.sigmoid(z + bout_ref[...])


_tc_c = pl.pallas_call(
    _tc_c_body,
    out_shape=jax.ShapeDtypeStruct((N, 1), jnp.float32),
)


def kernel(x, edge_index, W1, b1, W2, b2, Wout, bout):
    ei = edge_index.astype(jnp.int32)
    src = jnp.concatenate([ei[0], jnp.zeros((E_PAD - E,), jnp.int32)])
    dst = jnp.concatenate([ei[1], jnp.full((E_PAD - E,), SINK, jnp.int32)])
    src3 = src.reshape(NW, CPW, CHUNK)
    dst3 = dst.reshape(NW, CPW, CHUNK)

    xw1 = _tc_mm(x, W1)
    degp = _deg_kernel(dst3)
    y1, dinv = _tc_a(xw1, degp)

    p1 = _agg_kernel(y1, src3, dst3)
    y2 = _tc_b(p1, y1, dinv, b1.reshape(1, HID), W2)

    p2 = _agg_kernel(y2, src3, dst3)
    out = _tc_c(p2, y2, dinv, b2.reshape(1, HID), Wout, bout.reshape(1, 1))
    return out


# speedup vs baseline: 1.9697x; 1.1316x over previous
import functools

import jax
import jax.numpy as jnp
from jax import lax
from jax.experimental import pallas as pl
from jax.experimental.pallas import tpu as pltpu
from jax.experimental.pallas import tpu_sc as plsc

N = 10000
E = 320000
HID = 32
NC, NS = 2, 16
NW = NC * NS
CHUNK = 128
CPW = -(-E // (NW * CHUNK))
EPW = CPW * CHUNK
E_PAD = NW * EPW
NPAD = 10240
RPT = NPAD // NS
SINK = N

_mesh = plsc.VectorSubcoreMesh(core_axis_name="c", subcore_axis_name="s")
_sc_params = pltpu.CompilerParams(use_tc_tiling_on_sc=False)


@functools.partial(
    pl.kernel,
    out_type=jax.ShapeDtypeStruct((NC, NPAD, 16), jnp.float32),
    mesh=_mesh,
    scratch_types=[
        pltpu.VMEM((CPW, CHUNK), jnp.int32),
        pltpu.VMEM((CHUNK, 16), jnp.float32),
        pltpu.VMEM_SHARED((NPAD, 16), jnp.float32),
    ],
    compiler_params=_sc_params,
)
def _deg_kernel(dst_hbm, out_hbm, dstbuf, buf, deg_sh):
    c = lax.axis_index("c")
    s = lax.axis_index("s")
    wid = s * NC + c
    pltpu.sync_copy(dst_hbm.at[wid], dstbuf)

    def _zfill(i, _):
        buf[i, :] = jnp.zeros((16,), jnp.float32)
        return 0
    lax.fori_loop(0, CHUNK, _zfill, 0)
    for k in range(RPT // CHUNK):
        pltpu.sync_copy(buf, deg_sh.at[pl.ds(s * RPT + k * CHUNK, CHUNK)])

    def _ofill(i, _):
        buf[i, :] = jnp.ones((16,), jnp.float32)
        return 0
    lax.fori_loop(0, CHUNK, _ofill, 0)
    plsc.subcore_barrier()

    def _scat(j, _):
        pltpu.sync_copy(buf, deg_sh.at[dstbuf.at[j]], add=True)
        return 0
    lax.fori_loop(0, CPW, _scat, 0)
    plsc.subcore_barrier()
    pltpu.sync_copy(deg_sh.at[pl.ds(s * RPT, RPT)],
                    out_hbm.at[c, pl.ds(s * RPT, RPT)])


@functools.partial(
    pl.kernel,
    out_type=jax.ShapeDtypeStruct((NC, NPAD, HID), jnp.float32),
    mesh=_mesh,
    scratch_types=[
        pltpu.VMEM((CPW, CHUNK), jnp.int32),
        pltpu.VMEM((CPW, CHUNK), jnp.int32),
        pltpu.VMEM((2, CHUNK, HID), jnp.float32),
        pltpu.VMEM((CHUNK, HID), jnp.float32),
        pltpu.VMEM((RPT, HID), jnp.float32),
        pltpu.VMEM_SHARED((NPAD, HID), jnp.float32),
        pltpu.VMEM_SHARED((NPAD, HID), jnp.float32),
        pltpu.SemaphoreType.DMA,
        pltpu.SemaphoreType.DMA,
    ],
    compiler_params=_sc_params,
)
def _agg_kernel(y_hbm, src_hbm, dst_hbm, out_hbm,
                srcbuf, dstbuf, rows, zbuf, ybuf, y_sh, agg_sh, sem0, sem1):
    c = lax.axis_index("c")
    s = lax.axis_index("s")
    wid = s * NC + c
    pltpu.sync_copy(src_hbm.at[wid], srcbuf)
    pltpu.sync_copy(dst_hbm.at[wid], dstbuf)

    def _zfill(i, _):
        zbuf[i, pl.ds(0, 16)] = jnp.zeros((16,), jnp.float32)
        zbuf[i, pl.ds(16, 16)] = jnp.zeros((16,), jnp.float32)
        return 0
    lax.fori_loop(0, CHUNK, _zfill, 0)
    for k in range(RPT // CHUNK):
        pltpu.sync_copy(zbuf, agg_sh.at[pl.ds(s * RPT + k * CHUNK, CHUNK)])
    pltpu.sync_copy(y_hbm.at[pl.ds(s * RPT, RPT)], ybuf)
    pltpu.sync_copy(ybuf, y_sh.at[pl.ds(s * RPT, RPT)])
    plsc.subcore_barrier()

    gsem = (sem0, sem1)

    def _g(j, b):
        pltpu.async_copy(y_sh.at[srcbuf.at[j]], rows.at[b], gsem[b])

    def _wg(j, b):
        pltpu.make_async_copy(y_sh.at[srcbuf.at[j]], rows.at[b],
                              gsem[b]).wait()

    T2 = CPW // 2
    _g(0, 0)

    def _step(t, _):
        j0 = 2 * t
        _g(j0 + 1, 1)
        _wg(j0, 0)
        pltpu.sync_copy(rows.at[0], agg_sh.at[dstbuf.at[j0]], add=True)
        _g(j0 + 2, 0)
        _wg(j0 + 1, 1)
        pltpu.sync_copy(rows.at[1], agg_sh.at[dstbuf.at[j0 + 1]], add=True)
        return 0
    lax.fori_loop(0, T2, _step, 0)
    _wg(CPW - 1, 0)
    pltpu.sync_copy(rows.at[0], agg_sh.at[dstbuf.at[CPW - 1]], add=True)
    plsc.subcore_barrier()
    pltpu.sync_copy(agg_sh.at[pl.ds(s * RPT, RPT)],
                    out_hbm.at[c, pl.ds(s * RPT, RPT)])


def _tc_mm_body(x_ref, w1_ref, xw_ref):
    xw_ref[...] = jnp.dot(x_ref[...], w1_ref[...],
                          preferred_element_type=jnp.float32)


_tc_mm = pl.pallas_call(
    _tc_mm_body,
    out_shape=jax.ShapeDtypeStruct((N, HID), jnp.float32),
)


def _tc_a_body(xw_ref, degp_ref, y1_ref, dinv_ref):
    deg = degp_ref[0, :N, 0:1] + degp_ref[1, :N, 0:1] + 1.0
    dinv = lax.rsqrt(deg)
    y1_ref[pl.ds(0, N)] = xw_ref[...] * dinv
    y1_ref[pl.ds(N, NPAD - N)] = jnp.zeros((NPAD - N, HID), jnp.float32)
    dinv_ref[...] = dinv


_tc_a = pl.pallas_call(
    _tc_a_body,
    out_shape=[jax.ShapeDtypeStruct((NPAD, HID), jnp.float32),
               jax.ShapeDtypeStruct((N, 1), jnp.float32)],
)


def _tc_b_body(p_ref, y1_ref, dinv_ref, b1_ref, w2_ref, y2_ref):
    dinv = dinv_ref[...]
    agg = p_ref[0, :N, :] + p_ref[1, :N, :] + y1_ref[pl.ds(0, N)]
    h1 = jnp.maximum(dinv * agg + b1_ref[...], 0.0)
    xw2 = jnp.dot(h1, w2_ref[...], preferred_element_type=jnp.float32)
    y2_ref[pl.ds(0, N)] = xw2 * dinv
    y2_ref[pl.ds(N, NPAD - N)] = jnp.zeros((NPAD - N, HID), jnp.float32)


_tc_b = pl.pallas_call(
    _tc_b_body,
    out_shape=jax.ShapeDtypeStruct((NPAD, HID), jnp.float32),
)


def _tc_c_body(p_ref, y2_ref, dinv_ref, b2_ref, wout_ref, bout_ref,
               out_ref):
    dinv = dinv_ref[...]
    agg = p_ref[0, :N, :] + p_ref[1, :N, :] + y2_ref[pl.ds(0, N)]
    h2 = jnp.maximum(dinv * agg + b2_ref[...], 0.0)
    z = jnp.dot(h2, wout_ref[...], preferred_element_type=jnp.float32)
    out_ref[...] = jax.nn.sigmoid(z + bout_ref[...])


_tc_c = pl.pallas_call(
    _tc_c_body,
    out_shape=jax.ShapeDtypeStruct((N, 1), jnp.float32),
)


def kernel(x, edge_index, W1, b1, W2, b2, Wout, bout):
    ei = edge_index.astype(jnp.int32)
    src = jnp.concatenate([ei[0], jnp.zeros((E_PAD - E,), jnp.int32)])
    dst = jnp.concatenate([ei[1], jnp.full((E_PAD - E,), SINK, jnp.int32)])
    src3 = src.reshape(NW, CPW, CHUNK)
    dst3 = dst.reshape(NW, CPW, CHUNK)

    xw1 = _tc_mm(x, W1)
    degp = _deg_kernel(dst3)
    y1, dinv = _tc_a(xw1, degp)

    p1 = _agg_kernel(y1, src3, dst3)
    y2 = _tc_b(p1, y1, dinv, b1.reshape(1, HID), W2)

    p2 = _agg_kernel(y2, src3, dst3)
    out = _tc_c(p2, y2, dinv, b2.reshape(1, HID), Wout, bout.reshape(1, 1))
    return out


# speedup vs baseline: 1.9970x; 1.0138x over previous
import functools

import jax
import jax.numpy as jnp
from jax import lax
from jax.experimental import pallas as pl
from jax.experimental.pallas import tpu as pltpu
from jax.experimental.pallas import tpu_sc as plsc

N = 10000
E = 320000
HID = 32
NC, NS = 2, 16
NW = NC * NS
CHUNK = 128
NBUF = 4
CPW = NBUF * (-(-E // (NW * CHUNK * NBUF)))
EPW = CPW * CHUNK
E_PAD = NW * EPW
NPAD = 10240
RPT = NPAD // NS
SINK = N

_mesh = plsc.VectorSubcoreMesh(core_axis_name="c", subcore_axis_name="s")
_sc_params = pltpu.CompilerParams(use_tc_tiling_on_sc=False)


@functools.partial(
    pl.kernel,
    out_type=jax.ShapeDtypeStruct((NC, NPAD, 16), jnp.float32),
    mesh=_mesh,
    scratch_types=[
        pltpu.VMEM((CPW, CHUNK), jnp.int32),
        pltpu.VMEM((CHUNK, 16), jnp.float32),
        pltpu.VMEM_SHARED((NPAD, 16), jnp.float32),
        pltpu.SemaphoreType.DMA,
    ],
    compiler_params=_sc_params,
)
def _deg_kernel(dst_hbm, out_hbm, dstbuf, buf, deg_sh, ssem):
    c = lax.axis_index("c")
    s = lax.axis_index("s")
    wid = s * NC + c
    pltpu.sync_copy(dst_hbm.at[wid], dstbuf)

    def _zfill(i, _):
        buf[i, :] = jnp.zeros((16,), jnp.float32)
        return 0
    lax.fori_loop(0, CHUNK, _zfill, 0)
    for k in range(RPT // CHUNK):
        pltpu.sync_copy(buf, deg_sh.at[pl.ds(s * RPT + k * CHUNK, CHUNK)])

    def _ofill(i, _):
        buf[i, :] = jnp.ones((16,), jnp.float32)
        return 0
    lax.fori_loop(0, CHUNK, _ofill, 0)
    plsc.subcore_barrier()

    def _scat(j, _):
        pltpu.async_copy(buf, deg_sh.at[dstbuf.at[j]], ssem, add=True)
        return 0
    lax.fori_loop(0, CPW, _scat, 0)

    def _drain(j, _):
        pltpu.make_async_copy(buf, deg_sh.at[dstbuf.at[0]], ssem).wait()
        return 0
    lax.fori_loop(0, CPW, _drain, 0)
    plsc.subcore_barrier()
    pltpu.sync_copy(deg_sh.at[pl.ds(s * RPT, RPT)],
                    out_hbm.at[c, pl.ds(s * RPT, RPT)])


@functools.partial(
    pl.kernel,
    out_type=jax.ShapeDtypeStruct((NC, NPAD, HID), jnp.float32),
    mesh=_mesh,
    scratch_types=[
        pltpu.VMEM((CPW, CHUNK), jnp.int32),
        pltpu.VMEM((CPW, CHUNK), jnp.int32),
        pltpu.VMEM((NBUF, CHUNK, HID), jnp.float32),
        pltpu.VMEM((CHUNK, HID), jnp.float32),
        pltpu.VMEM((RPT, HID), jnp.float32),
        pltpu.VMEM_SHARED((NPAD, HID), jnp.float32),
        pltpu.VMEM_SHARED((NPAD, HID), jnp.float32),
    ] + [pltpu.SemaphoreType.DMA] * (2 * NBUF),
    compiler_params=_sc_params,
)
def _agg_kernel(y_hbm, src_hbm, dst_hbm, out_hbm,
                srcbuf, dstbuf, rows, zbuf, ybuf, y_sh, agg_sh, *sems):
    c = lax.axis_index("c")
    s = lax.axis_index("s")
    wid = s * NC + c
    pltpu.sync_copy(src_hbm.at[wid], srcbuf)
    pltpu.sync_copy(dst_hbm.at[wid], dstbuf)

    def _zfill(i, _):
        zbuf[i, pl.ds(0, 16)] = jnp.zeros((16,), jnp.float32)
        zbuf[i, pl.ds(16, 16)] = jnp.zeros((16,), jnp.float32)
        return 0
    lax.fori_loop(0, CHUNK, _zfill, 0)
    for k in range(RPT // CHUNK):
        pltpu.sync_copy(zbuf, agg_sh.at[pl.ds(s * RPT + k * CHUNK, CHUNK)])
    pltpu.sync_copy(y_hbm.at[pl.ds(s * RPT, RPT)], ybuf)
    pltpu.sync_copy(ybuf, y_sh.at[pl.ds(s * RPT, RPT)])
    plsc.subcore_barrier()

    gsem = sems[:NBUF]
    ssem = sems[NBUF:]

    def _g(j, b):
        pltpu.async_copy(y_sh.at[srcbuf.at[j]], rows.at[b], gsem[b])

    def _wg(j, b):
        pltpu.make_async_copy(y_sh.at[srcbuf.at[j]], rows.at[b],
                              gsem[b]).wait()

    def _s(j, b):
        pltpu.async_copy(rows.at[b], agg_sh.at[dstbuf.at[j]], ssem[b],
                         add=True)

    def _ws(b):
        pltpu.make_async_copy(rows.at[b], agg_sh.at[dstbuf.at[0]],
                              ssem[b]).wait()

    T = CPW // NBUF
    _g(0, 0)
    _g(1, 1)

    def _step(t, _):
        for b in range(NBUF):
            bg = (b + 2) % NBUF
            if b < 2:
                @pl.when(t > 0)
                def _():
                    _ws(bg)
                _g(NBUF * t + b + 2, bg)
            else:
                _ws(bg)

                @pl.when(t < T - 1)
                def _():
                    _g(NBUF * t + b + 2, bg)
            _wg(NBUF * t + b, b)
            _s(NBUF * t + b, b)
        return 0
    lax.fori_loop(0, T, _step, 0)
    _ws(2)
    _ws(3)
    plsc.subcore_barrier()
    pltpu.sync_copy(agg_sh.at[pl.ds(s * RPT, RPT)],
                    out_hbm.at[c, pl.ds(s * RPT, RPT)])


def _tc_mm_body(x_ref, w1_ref, xw_ref):
    xw_ref[...] = jnp.dot(x_ref[...], w1_ref[...],
                          preferred_element_type=jnp.float32)


_tc_mm = pl.pallas_call(
    _tc_mm_body,
    out_shape=jax.ShapeDtypeStruct((N, HID), jnp.float32),
)


def _tc_a_body(xw_ref, degp_ref, y1_ref, dinv_ref):
    deg = degp_ref[0, :N, 0:1] + degp_ref[1, :N, 0:1] + 1.0
    dinv = lax.rsqrt(deg)
    y1_ref[pl.ds(0, N)] = xw_ref[...] * dinv
    y1_ref[pl.ds(N, NPAD - N)] = jnp.zeros((NPAD - N, HID), jnp.float32)
    dinv_ref[...] = dinv


_tc_a = pl.pallas_call(
    _tc_a_body,
    out_shape=[jax.ShapeDtypeStruct((NPAD, HID), jnp.float32),
               jax.ShapeDtypeStruct((N, 1), jnp.float32)],
)


def _tc_b_body(p_ref, y1_ref, dinv_ref, b1_ref, w2_ref, y2_ref):
    dinv = dinv_ref[...]
    agg = p_ref[0, :N, :] + p_ref[1, :N, :] + y1_ref[pl.ds(0, N)]
    h1 = jnp.maximum(dinv * agg + b1_ref[...], 0.0)
    xw2 = jnp.dot(h1, w2_ref[...], preferred_element_type=jnp.float32)
    y2_ref[pl.ds(0, N)] = xw2 * dinv
    y2_ref[pl.ds(N, NPAD - N)] = jnp.zeros((NPAD - N, HID), jnp.float32)


_tc_b = pl.pallas_call(
    _tc_b_body,
    out_shape=jax.ShapeDtypeStruct((NPAD, HID), jnp.float32),
)


def _tc_c_body(p_ref, y2_ref, dinv_ref, b2_ref, wout_ref, bout_ref,
               out_ref):
    dinv = dinv_ref[...]
    agg = p_ref[0, :N, :] + p_ref[1, :N, :] + y2_ref[pl.ds(0, N)]
    h2 = jnp.maximum(dinv * agg + b2_ref[...], 0.0)
    z = jnp.dot(h2, wout_ref[...], preferred_element_type=jnp.float32)
    out_ref[...] = jax.nn.sigmoid(z + bout_ref[...])


_tc_c = pl.pallas_call(
    _tc_c_body,
    out_shape=jax.ShapeDtypeStruct((N, 1), jnp.float32),
)


def kernel(x, edge_index, W1, b1, W2, b2, Wout, bout):
    ei = edge_index.astype(jnp.int32)
    src = jnp.concatenate([ei[0], jnp.zeros((E_PAD - E,), jnp.int32)])
    dst = jnp.concatenate([ei[1], jnp.full((E_PAD - E,), SINK, jnp.int32)])
    src3 = src.reshape(NW, CPW, CHUNK)
    dst3 = dst.reshape(NW, CPW, CHUNK)

    xw1 = _tc_mm(x, W1)
    degp = _deg_kernel(dst3)
    y1, dinv = _tc_a(xw1, degp)

    p1 = _agg_kernel(y1, src3, dst3)
    y2 = _tc_b(p1, y1, dinv, b1.reshape(1, HID), W2)

    p2 = _agg_kernel(y2, src3, dst3)
    out = _tc_c(p2, y2, dinv, b2.reshape(1, HID), Wout, bout.reshape(1, 1))
    return out


# speedup vs baseline: 2.0568x; 1.0300x over previous
import functools

import jax
import jax.numpy as jnp
from jax import lax
from jax.experimental import pallas as pl
from jax.experimental.pallas import tpu as pltpu
from jax.experimental.pallas import tpu_sc as plsc

N = 10000
E = 320000
HID = 32
NC, NS = 2, 16
NW = NC * NS
CHUNK = 128
NBUF = 4
CPW = NBUF * (-(-E // (NW * CHUNK * NBUF)))
EPW = CPW * CHUNK
E_PAD = NW * EPW
NPAD = 10240
RPT = NPAD // NS
SINK = N

_mesh = plsc.VectorSubcoreMesh(core_axis_name="c", subcore_axis_name="s")
_sc_params = pltpu.CompilerParams(use_tc_tiling_on_sc=False)


@functools.partial(
    pl.kernel,
    out_type=jax.ShapeDtypeStruct((NC, NPAD, 16), jnp.float32),
    mesh=_mesh,
    scratch_types=[
        pltpu.VMEM((CPW, CHUNK), jnp.int32),
        pltpu.VMEM((CHUNK, 16), jnp.float32),
        pltpu.VMEM_SHARED((NPAD, 16), jnp.float32),
        pltpu.SemaphoreType.DMA,
    ],
    compiler_params=_sc_params,
)
def _deg_kernel(dst_hbm, out_hbm, dstbuf, buf, deg_sh, ssem):
    c = lax.axis_index("c")
    s = lax.axis_index("s")
    wid = s * NC + c
    pltpu.sync_copy(dst_hbm.at[wid], dstbuf)

    def _zfill(i, _):
        buf[i, :] = jnp.zeros((16,), jnp.float32)
        return 0
    lax.fori_loop(0, CHUNK, _zfill, 0)
    for k in range(RPT // CHUNK):
        pltpu.sync_copy(buf, deg_sh.at[pl.ds(s * RPT + k * CHUNK, CHUNK)])

    def _ofill(i, _):
        buf[i, :] = jnp.ones((16,), jnp.float32)
        return 0
    lax.fori_loop(0, CHUNK, _ofill, 0)
    plsc.subcore_barrier()

    def _scat(j, _):
        pltpu.async_copy(buf, deg_sh.at[dstbuf.at[j]], ssem, add=True)
        return 0
    lax.fori_loop(0, CPW, _scat, 0)

    def _drain(j, _):
        pltpu.make_async_copy(buf, deg_sh.at[dstbuf.at[0]], ssem).wait()
        return 0
    lax.fori_loop(0, CPW, _drain, 0)
    plsc.subcore_barrier()
    pltpu.sync_copy(deg_sh.at[pl.ds(s * RPT, RPT)],
                    out_hbm.at[c, pl.ds(s * RPT, RPT)])


@functools.partial(
    pl.kernel,
    out_type=jax.ShapeDtypeStruct((NC, NPAD, HID), jnp.float32),
    mesh=_mesh,
    scratch_types=[
        pltpu.VMEM((CPW, CHUNK), jnp.int32),
        pltpu.VMEM((CPW, CHUNK), jnp.int32),
        pltpu.VMEM((NBUF, CHUNK, HID), jnp.float32),
        pltpu.VMEM((CHUNK, HID), jnp.float32),
        pltpu.VMEM((RPT, HID), jnp.float32),
        pltpu.VMEM_SHARED((NPAD, HID), jnp.float32),
        pltpu.VMEM_SHARED((NPAD, HID), jnp.float32),
    ] + [pltpu.SemaphoreType.DMA] * (2 * NBUF),
    compiler_params=_sc_params,
)
def _agg_kernel(y_hbm, src_hbm, dst_hbm, out_hbm,
                srcbuf, dstbuf, rows, zbuf, ybuf, y_sh, agg_sh, *sems):
    gsem = sems[:NBUF]
    ssem = sems[NBUF:]
    c = lax.axis_index("c")
    s = lax.axis_index("s")
    wid = s * NC + c
    pltpu.async_copy(src_hbm.at[wid], srcbuf, gsem[0])
    pltpu.async_copy(dst_hbm.at[wid], dstbuf, gsem[1])
    pltpu.async_copy(y_hbm.at[pl.ds(s * RPT, RPT)], ybuf, gsem[2])

    def _zfill(i, _):
        zbuf[i, pl.ds(0, 16)] = jnp.zeros((16,), jnp.float32)
        zbuf[i, pl.ds(16, 16)] = jnp.zeros((16,), jnp.float32)
        return 0
    lax.fori_loop(0, CHUNK, _zfill, 0)
    for k in range(RPT // CHUNK):
        pltpu.sync_copy(zbuf, agg_sh.at[pl.ds(s * RPT + k * CHUNK, CHUNK)])
    pltpu.make_async_copy(src_hbm.at[wid], srcbuf, gsem[0]).wait()
    pltpu.make_async_copy(dst_hbm.at[wid], dstbuf, gsem[1]).wait()
    pltpu.make_async_copy(y_hbm.at[pl.ds(s * RPT, RPT)], ybuf,
                          gsem[2]).wait()
    pltpu.sync_copy(ybuf, y_sh.at[pl.ds(s * RPT, RPT)])
    plsc.subcore_barrier()

    def _g(j, b):
        pltpu.async_copy(y_sh.at[srcbuf.at[j]], rows.at[b], gsem[b])

    def _wg(j, b):
        pltpu.make_async_copy(y_sh.at[srcbuf.at[j]], rows.at[b],
                              gsem[b]).wait()

    def _s(j, b):
        pltpu.async_copy(rows.at[b], agg_sh.at[dstbuf.at[j]], ssem[b],
                         add=True)

    def _ws(b):
        pltpu.make_async_copy(rows.at[b], agg_sh.at[dstbuf.at[0]],
                              ssem[b]).wait()

    T = CPW // NBUF
    _g(0, 0)
    _g(1, 1)

    def _step(t, _):
        for b in range(NBUF):
            bg = (b + 2) % NBUF
            if b < 2:
                @pl.when(t > 0)
                def _():
                    _ws(bg)
                _g(NBUF * t + b + 2, bg)
            else:
                _ws(bg)

                @pl.when(t < T - 1)
                def _():
                    _g(NBUF * t + b + 2, bg)
            _wg(NBUF * t + b, b)
            _s(NBUF * t + b, b)
        return 0
    lax.fori_loop(0, T, _step, 0)
    _ws(2)
    _ws(3)
    plsc.subcore_barrier()
    pltpu.sync_copy(agg_sh.at[pl.ds(s * RPT, RPT)],
                    out_hbm.at[c, pl.ds(s * RPT, RPT)])


def _tc_mm_body(x_ref, w1_ref, xw_ref):
    xw_ref[...] = jnp.dot(x_ref[...], w1_ref[...],
                          preferred_element_type=jnp.float32)


_tc_mm = pl.pallas_call(
    _tc_mm_body,
    out_shape=jax.ShapeDtypeStruct((N, HID), jnp.float32),
)


def _tc_a_body(xw_ref, degp_ref, y1_ref, dinv_ref):
    deg = degp_ref[0, :N, 0:1] + degp_ref[1, :N, 0:1] + 1.0
    dinv = lax.rsqrt(deg)
    y1_ref[pl.ds(0, N)] = xw_ref[...] * dinv
    y1_ref[pl.ds(N, NPAD - N)] = jnp.zeros((NPAD - N, HID), jnp.float32)
    dinv_ref[...] = dinv


_tc_a = pl.pallas_call(
    _tc_a_body,
    out_shape=[jax.ShapeDtypeStruct((NPAD, HID), jnp.float32),
               jax.ShapeDtypeStruct((N, 1), jnp.float32)],
)


def _tc_b_body(p_ref, y1_ref, dinv_ref, b1_ref, w2_ref, y2_ref):
    dinv = dinv_ref[...]
    agg = p_ref[0, :N, :] + p_ref[1, :N, :] + y1_ref[pl.ds(0, N)]
    h1 = jnp.maximum(dinv * agg + b1_ref[...], 0.0)
    xw2 = jnp.dot(h1, w2_ref[...], preferred_element_type=jnp.float32)
    y2_ref[pl.ds(0, N)] = xw2 * dinv
    y2_ref[pl.ds(N, NPAD - N)] = jnp.zeros((NPAD - N, HID), jnp.float32)


_tc_b = pl.pallas_call(
    _tc_b_body,
    out_shape=jax.ShapeDtypeStruct((NPAD, HID), jnp.float32),
)


def _tc_c_body(p_ref, y2_ref, dinv_ref, b2_ref, wout_ref, bout_ref,
               out_ref):
    dinv = dinv_ref[...]
    agg = p_ref[0, :N, :] + p_ref[1, :N, :] + y2_ref[pl.ds(0, N)]
    h2 = jnp.maximum(dinv * agg + b2_ref[...], 0.0)
    z = jnp.dot(h2, wout_ref[...], preferred_element_type=jnp.float32)
    out_ref[...] = jax.nn.sigmoid(z + bout_ref[...])


_tc_c = pl.pallas_call(
    _tc_c_body,
    out_shape=jax.ShapeDtypeStruct((N, 1), jnp.float32),
)


def kernel(x, edge_index, W1, b1, W2, b2, Wout, bout):
    ei = edge_index.astype(jnp.int32)
    src = jnp.concatenate([ei[0], jnp.zeros((E_PAD - E,), jnp.int32)])
    dst = jnp.concatenate([ei[1], jnp.full((E_PAD - E,), SINK, jnp.int32)])
    src3 = src.reshape(NW, CPW, CHUNK)
    dst3 = dst.reshape(NW, CPW, CHUNK)

    xw1 = _tc_mm(x, W1)
    degp = _deg_kernel(dst3)
    y1, dinv = _tc_a(xw1, degp)

    p1 = _agg_kernel(y1, src3, dst3)
    y2 = _tc_b(p1, y1, dinv, b1.reshape(1, HID), W2)

    p2 = _agg_kernel(y2, src3, dst3)
    out = _tc_c(p2, y2, dinv, b2.reshape(1, HID), Wout, bout.reshape(1, 1))
    return out


# speedup vs baseline: 2.0610x; 1.0020x over previous
import functools

import jax
import jax.numpy as jnp
from jax import lax
from jax.experimental import pallas as pl
from jax.experimental.pallas import tpu as pltpu
from jax.experimental.pallas import tpu_sc as plsc

N = 10000
E = 320000
HID = 32
NC, NS = 2, 16
NW = NC * NS
CHUNK = 128
NBUF = 4
CPW = NBUF * (-(-E // (NW * CHUNK * NBUF)))
EPW = CPW * CHUNK
E_PAD = NW * EPW
NPAD = 10240
RPT = NPAD // NS
SINK = N

_mesh = plsc.VectorSubcoreMesh(core_axis_name="c", subcore_axis_name="s")
_sc_params = pltpu.CompilerParams(use_tc_tiling_on_sc=False,
                                 skip_device_barrier=True)


@functools.partial(
    pl.kernel,
    out_type=jax.ShapeDtypeStruct((NC, NPAD, 16), jnp.float32),
    mesh=_mesh,
    scratch_types=[
        pltpu.VMEM((CPW, CHUNK), jnp.int32),
        pltpu.VMEM((CHUNK, 16), jnp.float32),
        pltpu.VMEM_SHARED((NPAD, 16), jnp.float32),
        pltpu.SemaphoreType.DMA,
    ],
    compiler_params=_sc_params,
)
def _deg_kernel(dst_hbm, out_hbm, dstbuf, buf, deg_sh, ssem):
    c = lax.axis_index("c")
    s = lax.axis_index("s")
    wid = s * NC + c
    pltpu.sync_copy(dst_hbm.at[wid], dstbuf)

    def _zfill(i, _):
        buf[i, :] = jnp.zeros((16,), jnp.float32)
        return 0
    lax.fori_loop(0, CHUNK, _zfill, 0)
    for k in range(RPT // CHUNK):
        pltpu.sync_copy(buf, deg_sh.at[pl.ds(s * RPT + k * CHUNK, CHUNK)])

    def _ofill(i, _):
        buf[i, :] = jnp.ones((16,), jnp.float32)
        return 0
    lax.fori_loop(0, CHUNK, _ofill, 0)
    plsc.subcore_barrier()

    def _scat(j, _):
        pltpu.async_copy(buf, deg_sh.at[dstbuf.at[j]], ssem, add=True)
        return 0
    lax.fori_loop(0, CPW, _scat, 0)

    def _drain(j, _):
        pltpu.make_async_copy(buf, deg_sh.at[dstbuf.at[0]], ssem).wait()
        return 0
    lax.fori_loop(0, CPW, _drain, 0)
    plsc.subcore_barrier()
    pltpu.sync_copy(deg_sh.at[pl.ds(s * RPT, RPT)],
                    out_hbm.at[c, pl.ds(s * RPT, RPT)])


@functools.partial(
    pl.kernel,
    out_type=jax.ShapeDtypeStruct((NC, NPAD, HID), jnp.float32),
    mesh=_mesh,
    scratch_types=[
        pltpu.VMEM((CPW, CHUNK), jnp.int32),
        pltpu.VMEM((CPW, CHUNK), jnp.int32),
        pltpu.VMEM((NBUF, CHUNK, HID), jnp.float32),
        pltpu.VMEM((CHUNK, HID), jnp.float32),
        pltpu.VMEM((RPT, HID), jnp.float32),
        pltpu.VMEM_SHARED((NPAD, HID), jnp.float32),
        pltpu.VMEM_SHARED((NPAD, HID), jnp.float32),
    ] + [pltpu.SemaphoreType.DMA] * (2 * NBUF),
    compiler_params=_sc_params,
)
def _agg_kernel(y_hbm, src_hbm, dst_hbm, out_hbm,
                srcbuf, dstbuf, rows, zbuf, ybuf, y_sh, agg_sh, *sems):
    gsem = sems[:NBUF]
    ssem = sems[NBUF:]
    c = lax.axis_index("c")
    s = lax.axis_index("s")
    wid = s * NC + c
    pltpu.async_copy(src_hbm.at[wid], srcbuf, gsem[0])
    pltpu.async_copy(dst_hbm.at[wid], dstbuf, gsem[1])
    pltpu.async_copy(y_hbm.at[pl.ds(s * RPT, RPT)], ybuf, gsem[2])

    def _zfill(i, _):
        zbuf[i, pl.ds(0, 16)] = jnp.zeros((16,), jnp.float32)
        zbuf[i, pl.ds(16, 16)] = jnp.zeros((16,), jnp.float32)
        return 0
    lax.fori_loop(0, CHUNK, _zfill, 0)
    for k in range(RPT // CHUNK):
        pltpu.sync_copy(zbuf, agg_sh.at[pl.ds(s * RPT + k * CHUNK, CHUNK)])
    pltpu.make_async_copy(src_hbm.at[wid], srcbuf, gsem[0]).wait()
    pltpu.make_async_copy(dst_hbm.at[wid], dstbuf, gsem[1]).wait()
    pltpu.make_async_copy(y_hbm.at[pl.ds(s * RPT, RPT)], ybuf,
                          gsem[2]).wait()
    pltpu.sync_copy(ybuf, y_sh.at[pl.ds(s * RPT, RPT)])
    plsc.subcore_barrier()

    def _g(j, b):
        pltpu.async_copy(y_sh.at[srcbuf.at[j]], rows.at[b], gsem[b])

    def _wg(j, b):
        pltpu.make_async_copy(y_sh.at[srcbuf.at[j]], rows.at[b],
                              gsem[b]).wait()

    def _s(j, b):
        pltpu.async_copy(rows.at[b], agg_sh.at[dstbuf.at[j]], ssem[b],
                         add=True)

    def _ws(b):
        pltpu.make_async_copy(rows.at[b], agg_sh.at[dstbuf.at[0]],
                              ssem[b]).wait()

    T = CPW // NBUF
    _g(0, 0)
    _g(1, 1)

    def _step(t, _):
        for b in range(NBUF):
            bg = (b + 2) % NBUF
            if b < 2:
                @pl.when(t > 0)
                def _():
                    _ws(bg)
                _g(NBUF * t + b + 2, bg)
            else:
                _ws(bg)

                @pl.when(t < T - 1)
                def _():
                    _g(NBUF * t + b + 2, bg)
            _wg(NBUF * t + b, b)
            _s(NBUF * t + b, b)
        return 0
    lax.fori_loop(0, T, _step, 0)
    _ws(2)
    _ws(3)
    plsc.subcore_barrier()
    pltpu.sync_copy(agg_sh.at[pl.ds(s * RPT, RPT)],
                    out_hbm.at[c, pl.ds(s * RPT, RPT)])


def _tc_mm_body(x_ref, w1_ref, xw_ref):
    xw_ref[...] = jnp.dot(x_ref[...], w1_ref[...],
                          preferred_element_type=jnp.float32)


_tc_mm = pl.pallas_call(
    _tc_mm_body,
    out_shape=jax.ShapeDtypeStruct((N, HID), jnp.float32),
)


def _tc_a_body(xw_ref, degp_ref, y1_ref, dinv_ref):
    deg = degp_ref[0, :N, 0:1] + degp_ref[1, :N, 0:1] + 1.0
    dinv = lax.rsqrt(deg)
    y1_ref[pl.ds(0, N)] = xw_ref[...] * dinv
    y1_ref[pl.ds(N, NPAD - N)] = jnp.zeros((NPAD - N, HID), jnp.float32)
    dinv_ref[...] = dinv


_tc_a = pl.pallas_call(
    _tc_a_body,
    out_shape=[jax.ShapeDtypeStruct((NPAD, HID), jnp.float32),
               jax.ShapeDtypeStruct((N, 1), jnp.float32)],
)


def _tc_b_body(p_ref, y1_ref, dinv_ref, b1_ref, w2_ref, y2_ref):
    dinv = dinv_ref[...]
    agg = p_ref[0, :N, :] + p_ref[1, :N, :] + y1_ref[pl.ds(0, N)]
    h1 = jnp.maximum(dinv * agg + b1_ref[...], 0.0)
    xw2 = jnp.dot(h1, w2_ref[...], preferred_element_type=jnp.float32)
    y2_ref[pl.ds(0, N)] = xw2 * dinv
    y2_ref[pl.ds(N, NPAD - N)] = jnp.zeros((NPAD - N, HID), jnp.float32)


_tc_b = pl.pallas_call(
    _tc_b_body,
    out_shape=jax.ShapeDtypeStruct((NPAD, HID), jnp.float32),
)


def _tc_c_body(p_ref, y2_ref, dinv_ref, b2_ref, wout_ref, bout_ref,
               out_ref):
    dinv = dinv_ref[...]
    agg = p_ref[0, :N, :] + p_ref[1, :N, :] + y2_ref[pl.ds(0, N)]
    h2 = jnp.maximum(dinv * agg + b2_ref[...], 0.0)
    z = jnp.dot(h2, wout_ref[...], preferred_element_type=jnp.float32)
    out_ref[...] = jax.nn.sigmoid(z + bout_ref[...])


_tc_c = pl.pallas_call(
    _tc_c_body,
    out_shape=jax.ShapeDtypeStruct((N, 1), jnp.float32),
)


def kernel(x, edge_index, W1, b1, W2, b2, Wout, bout):
    ei = edge_index.astype(jnp.int32)
    src = jnp.concatenate([ei[0], jnp.zeros((E_PAD - E,), jnp.int32)])
    dst = jnp.concatenate([ei[1], jnp.full((E_PAD - E,), SINK, jnp.int32)])
    src3 = src.reshape(NW, CPW, CHUNK)
    dst3 = dst.reshape(NW, CPW, CHUNK)

    xw1 = _tc_mm(x, W1)
    degp = _deg_kernel(dst3)
    y1, dinv = _tc_a(xw1, degp)

    p1 = _agg_kernel(y1, src3, dst3)
    y2 = _tc_b(p1, y1, dinv, b1.reshape(1, HID), W2)

    p2 = _agg_kernel(y2, src3, dst3)
    out = _tc_c(p2, y2, dinv, b2.reshape(1, HID), Wout, bout.reshape(1, 1))
    return out


# speedup vs baseline: 2.0650x; 1.0019x over previous
import functools

import jax
import jax.numpy as jnp
from jax import lax
from jax.experimental import pallas as pl
from jax.experimental.pallas import tpu as pltpu
from jax.experimental.pallas import tpu_sc as plsc

N = 10000
E = 320000
HID = 32
NC, NS = 2, 16
NW = NC * NS
CHUNK = 128
ACH = 256
NBUF = 4
CPW = NBUF * (-(-E // (NW * CHUNK * NBUF)))
ACW = CPW * CHUNK // ACH
EPW = CPW * CHUNK
E_PAD = NW * EPW
NPAD = 10240
RPT = NPAD // NS
SINK = N

_mesh = plsc.VectorSubcoreMesh(core_axis_name="c", subcore_axis_name="s")
_sc_params = pltpu.CompilerParams(use_tc_tiling_on_sc=False,
                                 skip_device_barrier=True)


@functools.partial(
    pl.kernel,
    out_type=jax.ShapeDtypeStruct((NC, NPAD, 16), jnp.float32),
    mesh=_mesh,
    scratch_types=[
        pltpu.VMEM((CPW, CHUNK), jnp.int32),
        pltpu.VMEM((CHUNK, 16), jnp.float32),
        pltpu.VMEM_SHARED((NPAD, 16), jnp.float32),
        pltpu.SemaphoreType.DMA,
    ],
    compiler_params=_sc_params,
)
def _deg_kernel(dst_hbm, out_hbm, dstbuf, buf, deg_sh, ssem):
    c = lax.axis_index("c")
    s = lax.axis_index("s")
    wid = s * NC + c
    pltpu.sync_copy(dst_hbm.at[wid], dstbuf)

    def _zfill(i, _):
        buf[i, :] = jnp.zeros((16,), jnp.float32)
        return 0
    lax.fori_loop(0, CHUNK, _zfill, 0)
    for k in range(RPT // CHUNK):
        pltpu.sync_copy(buf, deg_sh.at[pl.ds(s * RPT + k * CHUNK, CHUNK)])

    def _ofill(i, _):
        buf[i, :] = jnp.ones((16,), jnp.float32)
        return 0
    lax.fori_loop(0, CHUNK, _ofill, 0)
    plsc.subcore_barrier()

    def _scat(j, _):
        pltpu.async_copy(buf, deg_sh.at[dstbuf.at[j]], ssem, add=True)
        return 0
    lax.fori_loop(0, CPW, _scat, 0)

    def _drain(j, _):
        pltpu.make_async_copy(buf, deg_sh.at[dstbuf.at[0]], ssem).wait()
        return 0
    lax.fori_loop(0, CPW, _drain, 0)
    plsc.subcore_barrier()
    pltpu.sync_copy(deg_sh.at[pl.ds(s * RPT, RPT)],
                    out_hbm.at[c, pl.ds(s * RPT, RPT)])


@functools.partial(
    pl.kernel,
    out_type=jax.ShapeDtypeStruct((NC, NPAD, HID), jnp.float32),
    mesh=_mesh,
    scratch_types=[
        pltpu.VMEM((ACW, ACH), jnp.int32),
        pltpu.VMEM((ACW, ACH), jnp.int32),
        pltpu.VMEM((NBUF, ACH, HID), jnp.float32),
        pltpu.VMEM((CHUNK, HID), jnp.float32),
        pltpu.VMEM((RPT, HID), jnp.float32),
        pltpu.VMEM_SHARED((NPAD, HID), jnp.float32),
        pltpu.VMEM_SHARED((NPAD, HID), jnp.float32),
    ] + [pltpu.SemaphoreType.DMA] * (2 * NBUF),
    compiler_params=_sc_params,
)
def _agg_kernel(y_hbm, src_hbm, dst_hbm, out_hbm,
                srcbuf, dstbuf, rows, zbuf, ybuf, y_sh, agg_sh, *sems):
    gsem = sems[:NBUF]
    ssem = sems[NBUF:]
    c = lax.axis_index("c")
    s = lax.axis_index("s")
    wid = s * NC + c
    pltpu.async_copy(src_hbm.at[wid], srcbuf, gsem[0])
    pltpu.async_copy(dst_hbm.at[wid], dstbuf, gsem[1])
    pltpu.async_copy(y_hbm.at[pl.ds(s * RPT, RPT)], ybuf, gsem[2])

    def _zfill(i, _):
        zbuf[i, pl.ds(0, 16)] = jnp.zeros((16,), jnp.float32)
        zbuf[i, pl.ds(16, 16)] = jnp.zeros((16,), jnp.float32)
        return 0
    lax.fori_loop(0, CHUNK, _zfill, 0)
    for k in range(RPT // CHUNK):
        pltpu.sync_copy(zbuf, agg_sh.at[pl.ds(s * RPT + k * CHUNK, CHUNK)])
    pltpu.make_async_copy(src_hbm.at[wid], srcbuf, gsem[0]).wait()
    pltpu.make_async_copy(dst_hbm.at[wid], dstbuf, gsem[1]).wait()
    pltpu.make_async_copy(y_hbm.at[pl.ds(s * RPT, RPT)], ybuf,
                          gsem[2]).wait()
    pltpu.sync_copy(ybuf, y_sh.at[pl.ds(s * RPT, RPT)])
    plsc.subcore_barrier()

    def _g(j, b):
        pltpu.async_copy(y_sh.at[srcbuf.at[j]], rows.at[b], gsem[b])

    def _wg(j, b):
        pltpu.make_async_copy(y_sh.at[srcbuf.at[j]], rows.at[b],
                              gsem[b]).wait()

    def _s(j, b):
        pltpu.async_copy(rows.at[b], agg_sh.at[dstbuf.at[j]], ssem[b],
                         add=True)

    def _ws(b):
        pltpu.make_async_copy(rows.at[b], agg_sh.at[dstbuf.at[0]],
                              ssem[b]).wait()

    T = ACW // NBUF
    _g(0, 0)
    _g(1, 1)

    def _step(t, _):
        for b in range(NBUF):
            bg = (b + 2) % NBUF
            if b < 2:
                @pl.when(t > 0)
                def _():
                    _ws(bg)
                _g(NBUF * t + b + 2, bg)
            else:
                _ws(bg)

                @pl.when(t < T - 1)
                def _():
                    _g(NBUF * t + b + 2, bg)
            _wg(NBUF * t + b, b)
            _s(NBUF * t + b, b)
        return 0
    lax.fori_loop(0, T, _step, 0)
    _ws(2)
    _ws(3)
    plsc.subcore_barrier()
    pltpu.sync_copy(agg_sh.at[pl.ds(s * RPT, RPT)],
                    out_hbm.at[c, pl.ds(s * RPT, RPT)])


def _tc_mm_body(x_ref, w1_ref, xw_ref):
    xw_ref[...] = jnp.dot(x_ref[...], w1_ref[...],
                          preferred_element_type=jnp.float32)


_tc_mm = pl.pallas_call(
    _tc_mm_body,
    out_shape=jax.ShapeDtypeStruct((N, HID), jnp.float32),
)


def _tc_a_body(xw_ref, degp_ref, y1_ref, dinv_ref):
    deg = degp_ref[0, :N, 0:1] + degp_ref[1, :N, 0:1] + 1.0
    dinv = lax.rsqrt(deg)
    y1_ref[pl.ds(0, N)] = xw_ref[...] * dinv
    y1_ref[pl.ds(N, NPAD - N)] = jnp.zeros((NPAD - N, HID), jnp.float32)
    dinv_ref[...] = dinv


_tc_a = pl.pallas_call(
    _tc_a_body,
    out_shape=[jax.ShapeDtypeStruct((NPAD, HID), jnp.float32),
               jax.ShapeDtypeStruct((N, 1), jnp.float32)],
)


def _tc_b_body(p_ref, y1_ref, dinv_ref, b1_ref, w2_ref, y2_ref):
    dinv = dinv_ref[...]
    agg = p_ref[0, :N, :] + p_ref[1, :N, :] + y1_ref[pl.ds(0, N)]
    h1 = jnp.maximum(dinv * agg + b1_ref[...], 0.0)
    xw2 = jnp.dot(h1, w2_ref[...], preferred_element_type=jnp.float32)
    y2_ref[pl.ds(0, N)] = xw2 * dinv
    y2_ref[pl.ds(N, NPAD - N)] = jnp.zeros((NPAD - N, HID), jnp.float32)


_tc_b = pl.pallas_call(
    _tc_b_body,
    out_shape=jax.ShapeDtypeStruct((NPAD, HID), jnp.float32),
)


def _tc_c_body(p_ref, y2_ref, dinv_ref, b2_ref, wout_ref, bout_ref,
               out_ref):
    dinv = dinv_ref[...]
    agg = p_ref[0, :N, :] + p_ref[1, :N, :] + y2_ref[pl.ds(0, N)]
    h2 = jnp.maximum(dinv * agg + b2_ref[...], 0.0)
    z = jnp.dot(h2, wout_ref[...], preferred_element_type=jnp.float32)
    out_ref[...] = jax.nn.sigmoid(z + bout_ref[...])


_tc_c = pl.pallas_call(
    _tc_c_body,
    out_shape=jax.ShapeDtypeStruct((N, 1), jnp.float32),
)


def kernel(x, edge_index, W1, b1, W2, b2, Wout, bout):
    ei = edge_index.astype(jnp.int32)
    src = jnp.concatenate([ei[0], jnp.zeros((E_PAD - E,), jnp.int32)])
    dst = jnp.concatenate([ei[1], jnp.full((E_PAD - E,), SINK, jnp.int32)])
    src3 = src.reshape(NW, ACW, ACH)
    dst3 = dst.reshape(NW, ACW, ACH)
    dst3d = dst.reshape(NW, CPW, CHUNK)

    xw1 = _tc_mm(x, W1)
    degp = _deg_kernel(dst3d)
    y1, dinv = _tc_a(xw1, degp)

    p1 = _agg_kernel(y1, src3, dst3)
    y2 = _tc_b(p1, y1, dinv, b1.reshape(1, HID), W2)

    p2 = _agg_kernel(y2, src3, dst3)
    out = _tc_c(p2, y2, dinv, b2.reshape(1, HID), Wout, bout.reshape(1, 1))
    return out
